# Initial kernel scaffold; baseline (speedup 1.0000x reference)
import jax
import jax.numpy as jnp
from jax.experimental import pallas as pl


def kernel(x, W, b):
    raise NotImplementedError("write your pallas kernel here")


# speedup vs baseline: 1.3130x; 1.3130x over previous
import jax
import jax.numpy as jnp
from jax.experimental import pallas as pl
from jax.experimental.pallas import tpu as pltpu


def _fused_block(x_ref, w_ref, b_ref, o_ref):
    x = x_ref[...]
    y = jax.lax.dot_general(
        x, w_ref[...], (((1,), (1,)), ((), ())),
        preferred_element_type=jnp.float32,
    )
    o_ref[...] = x + 0.1 * (y + b_ref[...])


def kernel(x, W, b):
    n, a = x.shape
    tm = 1000
    grid = (n // tm,)
    return pl.pallas_call(
        _fused_block,
        grid=grid,
        in_specs=[
            pl.BlockSpec((tm, a), lambda i: (i, 0)),
            pl.BlockSpec((a, a), lambda i: (0, 0)),
            pl.BlockSpec((1, a), lambda i: (0, 0)),
        ],
        out_specs=pl.BlockSpec((tm, a), lambda i: (i, 0)),
        out_shape=jax.ShapeDtypeStruct((n, a), jnp.float32),
        compiler_params=pltpu.CompilerParams(
            dimension_semantics=("parallel",),
        ),
    )(x, W, b.reshape(1, a))


# speedup vs baseline: 1.4373x; 1.0946x over previous
import jax
import jax.numpy as jnp
from jax.experimental import pallas as pl
from jax.experimental.pallas import tpu as pltpu


def _fused_block(x_ref, w_ref, b_ref, o_ref):
    x = x_ref[...]
    y = jax.lax.dot_general(
        x, w_ref[...], (((1,), (1,)), ((), ())),
        preferred_element_type=jnp.float32,
    )
    o_ref[...] = x + 0.1 * (y + b_ref[...])


def kernel(x, W, b):
    n, a = x.shape
    tm = 2000
    grid = (n // tm,)
    return pl.pallas_call(
        _fused_block,
        grid=grid,
        in_specs=[
            pl.BlockSpec((tm, a), lambda i: (i, 0)),
            pl.BlockSpec((a, a), lambda i: (0, 0)),
            pl.BlockSpec((1, a), lambda i: (0, 0)),
        ],
        out_specs=pl.BlockSpec((tm, a), lambda i: (i, 0)),
        out_shape=jax.ShapeDtypeStruct((n, a), jnp.float32),
        compiler_params=pltpu.CompilerParams(
            dimension_semantics=("parallel",),
        ),
    )(x, W, b.reshape(1, a))


# speedup vs baseline: 1.6659x; 1.1591x over previous
import jax
import jax.numpy as jnp
from jax.experimental import pallas as pl
from jax.experimental.pallas import tpu as pltpu


def _fused_block(x_ref, w_ref, b_ref, o_ref):
    x = x_ref[...]
    y = jax.lax.dot_general(
        x, w_ref[...], (((1,), (1,)), ((), ())),
        preferred_element_type=jnp.float32,
    )
    o_ref[...] = x + 0.1 * (y + b_ref[...])


def kernel(x, W, b):
    n, a = x.shape
    tm = 5000
    grid = (n // tm,)
    return pl.pallas_call(
        _fused_block,
        grid=grid,
        in_specs=[
            pl.BlockSpec((tm, a), lambda i: (i, 0)),
            pl.BlockSpec((a, a), lambda i: (0, 0)),
            pl.BlockSpec((1, a), lambda i: (0, 0)),
        ],
        out_specs=pl.BlockSpec((tm, a), lambda i: (i, 0)),
        out_shape=jax.ShapeDtypeStruct((n, a), jnp.float32),
        compiler_params=pltpu.CompilerParams(
            dimension_semantics=("parallel",),
        ),
    )(x, W, b.reshape(1, a))
